# Initial kernel scaffold; baseline (speedup 1.0000x reference)
import jax
import jax.numpy as jnp
from jax.experimental import pallas as pl


def kernel(x, token_embed, pos_table):
    raise NotImplementedError("write your pallas kernel here")


# speedup vs baseline: 1.8002x; 1.8002x over previous
import jax
import jax.numpy as jnp
from jax.experimental import pallas as pl


def _add_kernel(tok_ref, pos_ref, out_ref):
    out_ref[...] = tok_ref[...] + pos_ref[...]


def kernel(x, token_embed, pos_table):
    B, L, D = token_embed.shape
    L_BLK = 512
    grid = (L // L_BLK,)
    return pl.pallas_call(
        _add_kernel,
        grid=grid,
        in_specs=[
            pl.BlockSpec((B, L_BLK, D), lambda i: (0, i, 0)),
            pl.BlockSpec((L_BLK, D), lambda i: (i, 0)),
        ],
        out_specs=pl.BlockSpec((B, L_BLK, D), lambda i: (0, i, 0)),
        out_shape=jax.ShapeDtypeStruct((B, L, D), token_embed.dtype),
    )(token_embed, pos_table)


# speedup vs baseline: 1.8022x; 1.0011x over previous
import jax
import jax.numpy as jnp
from jax.experimental import pallas as pl


def _add_kernel(tok_ref, pos_ref, out_ref):
    out_ref[...] = tok_ref[...] + pos_ref[...]


def kernel(x, token_embed, pos_table):
    B, L, D = token_embed.shape
    L_BLK = 1024
    grid = (L // L_BLK,)
    return pl.pallas_call(
        _add_kernel,
        grid=grid,
        in_specs=[
            pl.BlockSpec((B, L_BLK, D), lambda i: (0, i, 0)),
            pl.BlockSpec((L_BLK, D), lambda i: (i, 0)),
        ],
        out_specs=pl.BlockSpec((B, L_BLK, D), lambda i: (0, i, 0)),
        out_shape=jax.ShapeDtypeStruct((B, L, D), token_embed.dtype),
    )(token_embed, pos_table)
